# speedup vs baseline: 51.4940x; 51.4940x over previous
import functools

import jax
import jax.numpy as jnp
from jax import lax
from jax.experimental import pallas as pl
from jax.experimental.pallas import tpu as pltpu
from jax.experimental.pallas import tpu_sc as plsc

_B, _N = 1, 6
_HI, _WI = 224, 480
_DOWN = 8
_DH, _DW = _HI // _DOWN, _WI // _DOWN
_D = 112
_S = 1
_C = 64
_K = _D + _S + _C
_XD, _YD = 200, 200

_P = _N * _D * _DH * _DW
_NW = 32
_ROWS = 276
_CHUNK = _ROWS * 128
_ACC_N = 40960
_TILE_SLICE = _ACC_N // 16


def _create_frustum():
    depth_grid = jnp.arange(2.0, 58.0, 0.5, dtype=jnp.float32)
    depth = jnp.broadcast_to(depth_grid[:, None, None], (_D, _DH, _DW))
    xg = jnp.broadcast_to(
        jnp.linspace(0.0, _WI - 1, _DW, dtype=jnp.float32)[None, None, :], (_D, _DH, _DW))
    yg = jnp.broadcast_to(
        jnp.linspace(0.0, _HI - 1, _DH, dtype=jnp.float32)[None, :, None], (_D, _DH, _DW))
    return jnp.stack((xg, yg, depth), -1)


def _geometry(intrinsics, extrinsics):
    rotation = extrinsics[..., :3, :3]
    translation = extrinsics[..., :3, 3]
    frustum = _create_frustum()
    points = frustum[None, None, ..., None]
    points = jnp.concatenate(
        (points[..., :2, :] * points[..., 2:3, :], points[..., 2:3, :]), axis=-2)
    combined = jnp.matmul(rotation, jnp.linalg.inv(intrinsics))
    pts = jnp.matmul(combined.reshape(_B, _N, 1, 1, 1, 3, 3), points).squeeze(-1)
    pts = pts + translation.reshape(_B, _N, 1, 1, 1, 3)
    return pts


def _tc_body(img_ref, gx_ref, gy_ref, gz_ref, pht_ref, pw_ref, wenc_ref,
             benc_ref, wdec_ref, depth_ref, seg_ref, vals_ref, idx_ref):
    img = img_ref[0]
    pht = pht_ref[...]
    pw = pw_ref[...]
    w = wenc_ref[...]
    b = benc_ref[...]
    wd = wdec_ref[...]

    pooled = []
    for c in range(3):
        t = jnp.dot(pht, img[c], preferred_element_type=jnp.float32,
                    precision=lax.Precision.HIGHEST)
        pooled.append(jnp.dot(t, pw, preferred_element_type=jnp.float32,
                              precision=lax.Precision.HIGHEST))

    feat = (w[0][:, None, None] * pooled[0][None]
            + w[1][:, None, None] * pooled[1][None]
            + w[2][:, None, None] * pooled[2][None]
            + b[:, :, None])

    logits = feat[:_D]
    m = jnp.max(logits, axis=0)
    e = jnp.exp(logits - m[None])
    s = jnp.sum(e, axis=0)
    depth = e / s[None]
    depth_ref[0] = depth
    seg_ref[0] = feat[_D:_D + _S]

    cf = feat[_D + _S:]
    proj = jnp.sum(cf * wd[:, 0][:, None, None], axis=0)
    vals = depth * proj[None]

    gx = ((gx_ref[0] - (-50.0)) / 0.5).astype(jnp.int32)
    gy = ((gy_ref[0] - (-50.0)) / 0.5).astype(jnp.int32)
    gz = ((gz_ref[0] - (-10.0)) / 20.0).astype(jnp.int32)
    kept = ((gx >= 0) & (gx < _XD) & (gy >= 0) & (gy < _YD)
            & (gz >= 0) & (gz < 1))
    vals_ref[0] = jnp.where(kept, vals, 0.0)
    idx_ref[0] = (jnp.clip(gx, 0, _XD - 1) * _YD
                  + jnp.clip(gy, 0, _YD - 1))


def _tc_encode(image, gx, gy, gz, pht, pw, wenc, benc, wdec):
    pt_blk = pl.BlockSpec((1, _D, _DH, _DW), lambda n: (n, 0, 0, 0))
    full = lambda shp: pl.BlockSpec(shp, lambda n: tuple(0 for _ in shp))
    return pl.pallas_call(
        _tc_body,
        grid=(_N,),
        in_specs=[
            pl.BlockSpec((1, 3, _HI, _WI), lambda n: (n, 0, 0, 0)),
            pt_blk, pt_blk, pt_blk,
            full((_DH, _HI)), full((_WI, _DW)),
            full((3, _K)), full((_K, 1)), full((_C, 1)),
        ],
        out_specs=[
            pt_blk,
            pl.BlockSpec((1, _S, _DH, _DW), lambda n: (n, 0, 0, 0)),
            pt_blk, pt_blk,
        ],
        out_shape=[
            jax.ShapeDtypeStruct((_N, _D, _DH, _DW), jnp.float32),
            jax.ShapeDtypeStruct((_N, _S, _DH, _DW), jnp.float32),
            jax.ShapeDtypeStruct((_N, _D, _DH, _DW), jnp.float32),
            jax.ShapeDtypeStruct((_N, _D, _DH, _DW), jnp.int32),
        ],
    )(image, gx, gy, gz, pht, pw, wenc, benc, wdec)


def _sc_body(vals_hbm, idx_hbm, zeros_hbm, out_hbm, vals_v, idx_v, acc_sh):
    cid = lax.axis_index("c")
    sid = lax.axis_index("s")
    wid = sid * 2 + cid

    pltpu.sync_copy(vals_hbm.at[wid], vals_v)
    pltpu.sync_copy(idx_hbm.at[wid], idx_v)
    pltpu.sync_copy(zeros_hbm, acc_sh.at[pl.ds(sid * _TILE_SLICE, _TILE_SLICE)])
    plsc.subcore_barrier()

    def step(j, carry):
        pltpu.sync_copy(vals_v.at[j], acc_sh.at[idx_v.at[j]], add=True)
        return carry

    lax.fori_loop(0, _ROWS, step, 0)
    plsc.subcore_barrier()

    sl = pl.ds(sid * _TILE_SLICE, _TILE_SLICE)
    pltpu.sync_copy(acc_sh.at[sl], out_hbm.at[cid, sl])


def _sc_scatter(vals3, idx3, zeros):
    mesh = plsc.VectorSubcoreMesh(core_axis_name="c", subcore_axis_name="s")
    fn = functools.partial(
        pl.kernel,
        mesh=mesh,
        out_type=jax.ShapeDtypeStruct((2, _ACC_N), jnp.float32),
        scratch_types=[
            pltpu.VMEM((_ROWS, 128), jnp.float32),
            pltpu.VMEM((_ROWS, 128), jnp.int32),
            pltpu.VMEM_SHARED((_ACC_N,), jnp.float32),
        ],
    )(_sc_body)
    return fn(vals3, idx3, zeros)


def kernel(image, intrinsics, extrinsics, W_enc, b_enc, W_dec, b_dec):
    geom = _geometry(intrinsics, extrinsics)[0]
    gx = geom[..., 0]
    gy = geom[..., 1]
    gz = geom[..., 2]

    pht = (jnp.repeat(jnp.eye(_DH, dtype=jnp.float32), _DOWN, axis=1) / _DOWN)
    pw = (jnp.repeat(jnp.eye(_DW, dtype=jnp.float32), _DOWN, axis=0) / _DOWN)

    depth, seg, vals, idx = _tc_encode(
        image.reshape(_N, 3, _HI, _WI), gx, gy, gz, pht, pw,
        W_enc, b_enc[:, None], W_dec)

    pad = _NW * _CHUNK - _P
    vals3 = jnp.pad(vals.reshape(-1), (0, pad)).reshape(_NW, _ROWS, 128)
    idx3 = jnp.pad(idx.reshape(-1), (0, pad)).reshape(_NW, _ROWS, 128)
    zeros = jnp.zeros((_TILE_SLICE,), jnp.float32)

    parts = _sc_scatter(vals3, idx3, zeros)
    bev = (parts[0, :_XD * _YD] + parts[1, :_XD * _YD]).reshape(1, 1, _XD, _YD)
    bev_output = bev + b_dec[None, :, None, None]
    return (bev_output, seg, depth)
